# baseline (speedup 1.0000x reference)
import jax
import jax.numpy as jnp
from jax import lax
from jax.experimental import pallas as pl
from jax.experimental.pallas import tpu as pltpu
from jax.experimental.pallas import tpu_sc as plsc

VOCAB = 1000000
SEQ_LEN = 200
EMBED = 32
BATCH = 4096

NC, NS = 2, 16
NW = NC * NS
SEQ_PER_W = BATCH // NW
SEQ_PER_CHUNK = 4
CHUNK = SEQ_PER_CHUNK * SEQ_LEN
NCHUNK = SEQ_PER_W // SEQ_PER_CHUNK
IDX_BLK = 32
NIBUF = 2
CHUNKS_PER_BLK = IDX_BLK // SEQ_PER_CHUNK

MID_ROWS = BATCH * SEQ_LEN * EMBED // 128
ROWS_PER_CHUNK = CHUNK * EMBED // 128
GROUPS = 128 // EMBED
L4 = SEQ_LEN // GROUPS


def _gather_body(inp_hbm, table_hbm, pos_hbm, out_hbm, idx0, idx1, pos_v,
                 rowa, rowb, wbufa, wbufb, i0, i1, g0, g1, w0, w1):
  idxs = [idx0, idx1]
  isem = [i0, i1]
  rows = [rowa, rowb]
  wbufs = [wbufa, wbufb]
  gsem = [g0, g1]
  wsem = [w0, w1]

  wid = lax.axis_index("s") * NC + lax.axis_index("c")
  wseq = wid * SEQ_PER_W
  wmid = wseq * SEQ_LEN * EMBED // 128

  def stage_idx(blk):
    return pltpu.async_copy(
        inp_hbm.at[pl.ds(wseq + blk * IDX_BLK, IDX_BLK)],
        idxs[blk % NIBUF], isem[blk % NIBUF])

  idesc = stage_idx(0)
  pltpu.sync_copy(pos_hbm, pos_v)
  idesc.wait()
  idesc = stage_idx(1)

  def start_chunk(g, b):
    ib = idxs[(g // CHUNKS_PER_BLK) % NIBUF]
    base = (g % CHUNKS_PER_BLK) * SEQ_PER_CHUNK
    descs = []
    for s in range(SEQ_PER_CHUNK):
      descs.append(pltpu.async_copy(
          table_hbm.at[ib.at[base + s]],
          rows[b].at[s],
          gsem[b]))
    return descs

  gdesc = [start_chunk(0, 0), start_chunk(1, 1)]
  wdesc = [None, None]

  for g in range(NCHUNK):
    b = g % 2
    for d in gdesc[b]:
      d.wait()
    if wdesc[b] is not None:
      wdesc[b].wait()
      wdesc[b] = None

    def add_repack(l4, carry, rbuf=rows[b], wbuf=wbufs[b]):
      for j in range(GROUPS):
        l = l4 * GROUPS + j
        p0 = pos_v[l, pl.ds(0, 16)]
        p1 = pos_v[l, pl.ds(16, 16)]
        for s in range(SEQ_PER_CHUNK):
          r = s * L4 + l4
          wbuf[r, pl.ds(j * EMBED, 16)] = rbuf[s, l, pl.ds(0, 16)] + p0
          wbuf[r, pl.ds(j * EMBED + 16, 16)] = rbuf[s, l, pl.ds(16, 16)] + p1
      return carry
    lax.fori_loop(0, L4, add_repack, 0)

    wdesc[b] = pltpu.async_copy(
        wbufs[b],
        out_hbm.at[pl.ds(wmid + g * ROWS_PER_CHUNK, ROWS_PER_CHUNK)],
        wsem[b])

    nxt = g + 2
    if nxt < NCHUNK:
      if nxt % CHUNKS_PER_BLK == 0:
        idesc.wait()
        nblk = nxt // CHUNKS_PER_BLK + 1
        if nblk * IDX_BLK < SEQ_PER_W:
          idesc = stage_idx(nblk)
      gdesc[b] = start_chunk(nxt, b)

  for b in range(2):
    if wdesc[b] is not None:
      wdesc[b].wait()


@jax.jit
def _run(inputs, token_table, pos_table):
  mesh = plsc.VectorSubcoreMesh(core_axis_name="c", subcore_axis_name="s")
  scratch = (
      [pltpu.VMEM((IDX_BLK, SEQ_LEN), jnp.int32) for _ in range(NIBUF)]
      + [pltpu.VMEM((SEQ_LEN, EMBED), jnp.float32)]
      + [pltpu.VMEM((SEQ_PER_CHUNK, SEQ_LEN, EMBED), jnp.float32)
         for _ in range(2)]
      + [pltpu.VMEM((ROWS_PER_CHUNK, 128), jnp.float32) for _ in range(2)]
      + [pltpu.SemaphoreType.DMA for _ in range(6)]
  )
  mid = pl.kernel(
      _gather_body,
      out_type=jax.ShapeDtypeStruct((MID_ROWS, 128), jnp.float32),
      mesh=mesh,
      scratch_types=scratch,
      compiler_params=pltpu.CompilerParams(use_tc_tiling_on_sc=False),
  )(inputs, token_table, pos_table)
  return mid.reshape(BATCH, SEQ_LEN, EMBED)


def kernel(inputs, token_table, pos_table):
  return _run(inputs, token_table, pos_table)


# speedup vs baseline: 1.0029x; 1.0029x over previous
import jax
import jax.numpy as jnp
from jax import lax
from jax.experimental import pallas as pl
from jax.experimental.pallas import tpu as pltpu
from jax.experimental.pallas import tpu_sc as plsc

VOCAB = 1000000
SEQ_LEN = 200
EMBED = 32
BATCH = 4096

NC, NS = 2, 16
NW = NC * NS
SEQ_PER_W = BATCH // NW
SEQ_PER_CHUNK = 4
CHUNK = SEQ_PER_CHUNK * SEQ_LEN
NCHUNK = SEQ_PER_W // SEQ_PER_CHUNK
NBUF = 4
IDX_BLK = 32
NIBUF = 2
CHUNKS_PER_BLK = IDX_BLK // SEQ_PER_CHUNK

def _gather_body(inp_hbm, table_hbm, pos_hbm, out_hbm, idx0, idx1, pos_v,
                 row0, row1, row2, row3, i0, i1, g0, g1, g2, g3,
                 w0, w1, w2, w3):
  idxs = [idx0, idx1]
  isem = [i0, i1]
  rows = [row0, row1, row2, row3]
  gsem = [g0, g1, g2, g3]
  wsem = [w0, w1, w2, w3]

  wid = lax.axis_index("s") * NC + lax.axis_index("c")
  wseq = wid * SEQ_PER_W

  def stage_idx(blk):
    return pltpu.async_copy(
        inp_hbm.at[pl.ds(wseq + blk * IDX_BLK, IDX_BLK)],
        idxs[blk % NIBUF], isem[blk % NIBUF])

  idesc = stage_idx(0)
  pltpu.sync_copy(pos_hbm, pos_v)
  idesc.wait()
  idesc = stage_idx(1)

  def start_chunk(g, b):
    ib = idxs[(g // CHUNKS_PER_BLK) % NIBUF]
    base = (g % CHUNKS_PER_BLK) * SEQ_PER_CHUNK
    descs = []
    for s in range(SEQ_PER_CHUNK):
      descs.append(pltpu.async_copy(
          table_hbm.at[ib.at[base + s]],
          rows[b].at[s],
          gsem[b]))
    return descs

  gdesc = [None] * NBUF
  wdesc = [None] * NBUF
  for b in range(2):
    gdesc[b] = start_chunk(b, b)

  for g in range(NCHUNK):
    b = g % NBUF
    for d in gdesc[b]:
      d.wait()

    def add_pos(i, carry, rbuf=rows[b]):
      for u in range(2):
        l = i * 2 + u
        p0 = pos_v[l, pl.ds(0, 16)]
        p1 = pos_v[l, pl.ds(16, 16)]
        for s in range(SEQ_PER_CHUNK):
          plsc.addupdate(rbuf.at[s, l, pl.ds(0, 16)], p0)
          plsc.addupdate(rbuf.at[s, l, pl.ds(16, 16)], p1)
      return carry
    lax.fori_loop(0, SEQ_LEN // 2, add_pos, 0)

    wdesc[b] = pltpu.async_copy(
        rows[b],
        out_hbm.at[pl.ds(wseq + g * SEQ_PER_CHUNK, SEQ_PER_CHUNK)],
        wsem[b])

    nxt = g + 2
    if nxt < NCHUNK:
      nb = nxt % NBUF
      if wdesc[nb] is not None:
        wdesc[nb].wait()
        wdesc[nb] = None
      if nxt % CHUNKS_PER_BLK == 0:
        idesc.wait()
        nblk = nxt // CHUNKS_PER_BLK + 1
        if nblk * IDX_BLK < SEQ_PER_W:
          idesc = stage_idx(nblk)
      gdesc[nb] = start_chunk(nxt, nb)

  for b in range(NBUF):
    if wdesc[b] is not None:
      wdesc[b].wait()


@jax.jit
def _run(inputs, token_table, pos_table):
  mesh = plsc.VectorSubcoreMesh(core_axis_name="c", subcore_axis_name="s")
  scratch = (
      [pltpu.VMEM((IDX_BLK, SEQ_LEN), jnp.int32) for _ in range(NIBUF)]
      + [pltpu.VMEM((SEQ_LEN, EMBED), jnp.float32)]
      + [pltpu.VMEM((SEQ_PER_CHUNK, SEQ_LEN, EMBED), jnp.float32)
         for _ in range(NBUF)]
      + [pltpu.SemaphoreType.DMA for _ in range(NIBUF + 2 * NBUF)]
  )
  return pl.kernel(
      _gather_body,
      out_type=jax.ShapeDtypeStruct((BATCH, SEQ_LEN, EMBED), jnp.float32),
      mesh=mesh,
      scratch_types=scratch,
      compiler_params=pltpu.CompilerParams(use_tc_tiling_on_sc=False),
  )(inputs, token_table, pos_table)


def kernel(inputs, token_table, pos_table):
  return _run(inputs, token_table, pos_table)
